# Initial kernel scaffold; baseline (speedup 1.0000x reference)
import jax
import jax.numpy as jnp
from jax.experimental import pallas as pl


def kernel(x, edge_index, W1, a_src1, a_dst1, b1, W2, a_src2, a_dst2, b2):
    raise NotImplementedError("write your pallas kernel here")


# speedup vs baseline: 43.7189x; 43.7189x over previous
import functools

import jax
import jax.numpy as jnp
from jax import lax
from jax.experimental import pallas as pl
from jax.experimental.pallas import tpu as pltpu
from jax.experimental.pallas import tpu_sc as plsc

N = 10000
NP = 10240
E = 320000
D = 128
H = 4
HID = 128
OUT = 128
NEG_SLOPE = 0.2

BN = 1024
CHUNK = 80
NSUB = 16
NCORE = 2
NPT = NP // NSUB
ZR = 40


def _tc1_body(x_ref, w1_ref, proj_ref, feats_ref, asad_ref):
    h = jnp.dot(x_ref[...], w1_ref[...], preferred_element_type=jnp.float32)
    for hh in range(H):
        feats_ref[hh] = h[:, hh * HID:(hh + 1) * HID]
    asad_ref[...] = lax.dot_general(
        proj_ref[...], h, (((1,), (1,)), ((), ())),
        preferred_element_type=jnp.float32)


def _tc1(x, W1, proj1):
    return pl.pallas_call(
        _tc1_body,
        grid=(NP // BN,),
        in_specs=[
            pl.BlockSpec((BN, D), lambda i: (i, 0)),
            pl.BlockSpec((D, H * HID), lambda i: (0, 0)),
            pl.BlockSpec((2 * H, H * HID), lambda i: (0, 0)),
        ],
        out_specs=[
            pl.BlockSpec((H, BN, HID), lambda i: (0, i, 0)),
            pl.BlockSpec((2 * H, BN), lambda i: (0, i)),
        ],
        out_shape=[
            jax.ShapeDtypeStruct((H, NP, HID), jnp.float32),
            jax.ShapeDtypeStruct((2 * H, NP), jnp.float32),
        ],
    )(x, W1, proj1)


def _tc2_body(acc_ref, den_ref, feats_ref, asad_ref, b1_ref, w2_ref,
              proj2_ref, h2_ref, asad2_ref):
    cols = []
    for hh in range(H):
        t = asad_ref[hh] + asad_ref[H + hh]
        p = jnp.exp(jnp.where(t >= 0, t, t * NEG_SLOPE))
        num = acc_ref[hh] + p[:, None] * feats_ref[hh]
        den = den_ref[hh] + p + 1e-16
        cols.append(num / den[:, None] + b1_ref[hh * HID:(hh + 1) * HID][None, :])
    act = jnp.maximum(jnp.concatenate(cols, axis=-1), 0.0)
    h2 = jnp.dot(act, w2_ref[...], preferred_element_type=jnp.float32)
    h2_ref[...] = h2
    asad2_ref[...] = lax.dot_general(
        proj2_ref[...], h2, (((1,), (1,)), ((), ())),
        preferred_element_type=jnp.float32)


def _tc2(acc1, den1, feats1, asad1, b1, W2, proj2):
    return pl.pallas_call(
        _tc2_body,
        grid=(NP // BN,),
        in_specs=[
            pl.BlockSpec((H, BN, HID), lambda i: (0, i, 0)),
            pl.BlockSpec((2 * H, BN), lambda i: (0, i)),
            pl.BlockSpec((H, BN, HID), lambda i: (0, i, 0)),
            pl.BlockSpec((2 * H, BN), lambda i: (0, i)),
            pl.BlockSpec((H * HID,), lambda i: (0,)),
            pl.BlockSpec((H * HID, OUT), lambda i: (0, 0)),
            pl.BlockSpec((2 * H, OUT), lambda i: (0, 0)),
        ],
        out_specs=[
            pl.BlockSpec((BN, OUT), lambda i: (i, 0)),
            pl.BlockSpec((2 * H, BN), lambda i: (0, i)),
        ],
        out_shape=[
            jax.ShapeDtypeStruct((NP, OUT), jnp.float32),
            jax.ShapeDtypeStruct((2 * H, NP), jnp.float32),
        ],
    )(acc1, den1, feats1, asad1, b1, W2, proj2)


def _tc3_body(acc_ref, den_ref, h2_ref, asad2_ref, b2_ref, out_ref):
    t = asad2_ref[0] + asad2_ref[1]
    p = jnp.exp(jnp.where(t >= 0, t, t * NEG_SLOPE))
    num = acc_ref[0] + acc_ref[1] + p[:, None] * h2_ref[...]
    den = den_ref[0] + den_ref[1] + p + 1e-16
    out_ref[...] = num / den[:, None] + b2_ref[...][None, :]


def _tc3(acc2, den2, h2, asad2, b2):
    return pl.pallas_call(
        _tc3_body,
        grid=(NP // BN,),
        in_specs=[
            pl.BlockSpec((NCORE, BN, OUT), lambda i: (0, i, 0)),
            pl.BlockSpec((2 * H, BN), lambda i: (0, i)),
            pl.BlockSpec((BN, OUT), lambda i: (i, 0)),
            pl.BlockSpec((2 * H, BN), lambda i: (0, i)),
            pl.BlockSpec((OUT,), lambda i: (0,)),
        ],
        out_specs=pl.BlockSpec((BN, OUT), lambda i: (i, 0)),
        out_shape=jax.ShapeDtypeStruct((NP, OUT), jnp.float32),
    )(acc2, den2, h2, asad2, b2)


_MESH = plsc.VectorSubcoreMesh(
    core_axis_name="c", subcore_axis_name="s", num_cores=NCORE,
    num_subcores=NSUB)

_BUF = [
    pltpu.VMEM((CHUNK,), jnp.int32),
    pltpu.VMEM((CHUNK,), jnp.int32),
    pltpu.VMEM((CHUNK,), jnp.int32),
    pltpu.VMEM((CHUNK,), jnp.float32),
    pltpu.VMEM((CHUNK, HID), jnp.float32),
    pltpu.VMEM((CHUNK,), jnp.int32),
    pltpu.SemaphoreType.DMA,
    pltpu.SemaphoreType.DMA,
    pltpu.SemaphoreType.DMA,
    pltpu.SemaphoreType.DMA,
]
_SC_SCRATCH = [
    pltpu.VMEM_SHARED((NP, HID), jnp.float32),
    pltpu.VMEM_SHARED((NP,), jnp.float32),
    pltpu.VMEM((NP,), jnp.float32),
    pltpu.VMEM((NP,), jnp.float32),
    pltpu.VMEM((ZR, HID), jnp.float32),
    pltpu.VMEM((NPT,), jnp.float32),
    _BUF,
    _BUF,
]


def _zero_bufs(zb, zden):
    def zrow(r, carry):
        for cc in range(HID // 16):
            zb[r, pl.ds(cc * 16, 16)] = jnp.zeros((16,), jnp.float32)
        return carry
    lax.fori_loop(0, ZR, zrow, 0)

    def zd(r, carry):
        zden[pl.ds(r * 16, 16)] = jnp.zeros((16,), jnp.float32)
        return carry
    lax.fori_loop(0, NPT // 16, zd, 0)


def _zero_acc(s, acc_s, den_s, zb, zden):
    base_n = s * NPT
    for z in range(NPT // ZR):
        pltpu.sync_copy(zb, acc_s.at[pl.ds(base_n + z * ZR, ZR)])
    pltpu.sync_copy(zden, den_s.at[pl.ds(s * NPT, NPT)])


def _edge_chunks(n_chunks, ebase, head_off, src_ref, dst_ref, feats_ref,
                 as_v, ad_v, bufs, acc_s, den_s):
    def idx_load(i, b, sync):
        srcb, dstb = bufs[b][0], bufs[b][1]
        isem = bufs[b][8]
        eb = ebase + i * CHUNK
        if sync:
            pltpu.sync_copy(src_ref.at[pl.ds(eb, CHUNK)], srcb)
            pltpu.sync_copy(dst_ref.at[pl.ds(eb, CHUNK)], dstb)
        else:
            pltpu.async_copy(src_ref.at[pl.ds(eb, CHUNK)], srcb, isem)
            pltpu.async_copy(dst_ref.at[pl.ds(eb, CHUNK)], dstb, isem)

    def stage_a(i, b, first):
        srcb, dstb, gidx, wv, rows, sdst, gsem, ssem, isem, dsem = bufs[b]
        if not first:
            pltpu.make_async_copy(src_ref.at[pl.ds(0, CHUNK)], srcb,
                                  isem).wait()
            pltpu.make_async_copy(dst_ref.at[pl.ds(0, CHUNK)], dstb,
                                  isem).wait()
            pltpu.make_async_copy(wv, den_s.at[sdst], dsem).wait()

        def wbody(kk, carry2):
            off = kk * 16
            sv = srcb[pl.ds(off, 16)]
            dv = dstb[pl.ds(off, 16)]
            t = plsc.load_gather(as_v, [sv]) + plsc.load_gather(ad_v, [dv])
            t = jnp.where(t >= 0, t, t * NEG_SLOPE)
            wv[pl.ds(off, 16)] = jnp.exp(t)
            gidx[pl.ds(off, 16)] = sv + head_off
            return carry2
        lax.fori_loop(0, CHUNK // 16, wbody, 0)
        if not first:
            pltpu.make_async_copy(rows, acc_s.at[sdst], ssem).wait()
        pltpu.async_copy(feats_ref.at[gidx], rows, gsem)

    def stage_b(i, b):
        srcb, dstb, gidx, wv, rows, sdst, gsem, ssem, isem, dsem = bufs[b]
        pltpu.make_async_copy(feats_ref.at[gidx], rows, gsem).wait()
        for kk in range(CHUNK // 16):
            sdst[pl.ds(kk * 16, 16)] = dstb[pl.ds(kk * 16, 16)]

        @pl.when(i + 2 < n_chunks)
        def _():
            idx_load(i + 2, b, False)

        def mbody(j, carry2):
            wj = plsc.load_gather(wv, [jnp.full((16,), j, jnp.int32)])
            for cc in range(HID // 16):
                rows[j, pl.ds(cc * 16, 16)] = rows[j, pl.ds(cc * 16, 16)] * wj
            return carry2
        lax.fori_loop(0, CHUNK, mbody, 0, unroll=4)

        pltpu.async_copy(rows, acc_s.at[sdst], ssem, add=True)
        pltpu.async_copy(wv, den_s.at[sdst], dsem, add=True)

    idx_load(0, 0, True)
    idx_load(1, 1, True)
    stage_a(0, 0, True)
    stage_a(1, 1, True)

    def loop_body(i2, carry):
        i = i2 * 2
        stage_b(i, 0)

        @pl.when(i + 2 < n_chunks)
        def _():
            stage_a(i + 2, 0, False)

        @pl.when(i + 1 < n_chunks)
        def _():
            stage_b(i + 1, 1)

            @pl.when(i + 3 < n_chunks)
            def _():
                stage_a(i + 3, 1, False)
        return carry
    lax.fori_loop(0, (n_chunks + 1) // 2, loop_body, 0)
    for b in range(2):
        srcb, dstb, gidx, wv, rows, sdst, gsem, ssem, isem, dsem = bufs[b]
        pltpu.make_async_copy(rows, acc_s.at[sdst], ssem).wait()
        pltpu.make_async_copy(wv, den_s.at[sdst], dsem).wait()


@functools.partial(
    pl.kernel,
    out_type=(jax.ShapeDtypeStruct((H * NP, HID), jnp.float32),
              jax.ShapeDtypeStruct((2 * H, NP), jnp.float32)),
    mesh=_MESH,
    scratch_types=_SC_SCRATCH,
    compiler_params=pltpu.CompilerParams(needs_layout_passes=False),
)
def _sc1(src_ref, dst_ref, asad_ref, feats_ref, accs_ref, dens_ref,
         acc_s, den_s, as_v, ad_v, zb, zden, buf0, buf1):
    c = lax.axis_index("c")
    s = lax.axis_index("s")
    _zero_bufs(zb, zden)
    eps = E // NSUB
    for hh in range(2):
        head = c * 2 + hh
        _zero_acc(s, acc_s, den_s, zb, zden)
        pltpu.sync_copy(asad_ref.at[head], as_v)
        pltpu.sync_copy(asad_ref.at[H + head], ad_v)
        plsc.subcore_barrier()
        _edge_chunks(eps // CHUNK, s * eps, head * NP, src_ref, dst_ref,
                     feats_ref, as_v, ad_v, (buf0, buf1), acc_s, den_s)
        plsc.subcore_barrier()
        base_n = s * NPT
        pltpu.sync_copy(acc_s.at[pl.ds(base_n, NPT)],
                        accs_ref.at[pl.ds(head * NP + base_n, NPT)])
        @pl.when(s == 0)
        def _():
            pltpu.sync_copy(den_s.at[pl.ds(0, NP)], dens_ref.at[head])
        plsc.subcore_barrier()


@functools.partial(
    pl.kernel,
    out_type=(jax.ShapeDtypeStruct((NCORE * NP, OUT), jnp.float32),
              jax.ShapeDtypeStruct((2 * H, NP), jnp.float32)),
    mesh=_MESH,
    scratch_types=_SC_SCRATCH,
    compiler_params=pltpu.CompilerParams(needs_layout_passes=False),
)
def _sc2(src_ref, dst_ref, asad_ref, feats_ref, accs_ref, dens_ref,
         acc_s, den_s, as_v, ad_v, zb, zden, buf0, buf1):
    c = lax.axis_index("c")
    s = lax.axis_index("s")
    _zero_bufs(zb, zden)
    eps = E // (NCORE * NSUB)
    _zero_acc(s, acc_s, den_s, zb, zden)
    pltpu.sync_copy(asad_ref.at[0], as_v)
    pltpu.sync_copy(asad_ref.at[1], ad_v)
    plsc.subcore_barrier()
    _edge_chunks(eps // CHUNK, (c * NSUB + s) * eps, 0, src_ref, dst_ref,
                 feats_ref, as_v, ad_v, (buf0, buf1), acc_s, den_s)
    plsc.subcore_barrier()
    base_n = s * NPT
    pltpu.sync_copy(acc_s.at[pl.ds(base_n, NPT)],
                    accs_ref.at[pl.ds(c * NP + base_n, NPT)])
    @pl.when(s == 0)
    def _():
        pltpu.sync_copy(den_s.at[pl.ds(0, NP)], dens_ref.at[c])


def kernel(x, edge_index, W1, a_src1, a_dst1, b1, W2, a_src2, a_dst2, b2):
    src = edge_index[0]
    dst = edge_index[1]
    x_p = jnp.pad(x, ((0, NP - N), (0, 0)))
    eye = jnp.eye(H, dtype=jnp.float32)
    proj1 = jnp.concatenate(
        [(eye[:, :, None] * a_src1[None, :, :]).reshape(H, H * HID),
         (eye[:, :, None] * a_dst1[None, :, :]).reshape(H, H * HID)], axis=0)
    proj2 = jnp.concatenate(
        [a_src2, a_dst2, jnp.zeros((2 * H - 2, OUT), jnp.float32)], axis=0)
    feats1, asad1 = _tc1(x_p, W1, proj1)
    acc1, den1 = _sc1(src, dst, asad1, feats1.reshape(H * NP, HID))
    h2, asad2 = _tc2(acc1.reshape(H, NP, HID), den1, feats1, asad1, b1, W2,
                     proj2)
    acc2, den2 = _sc2(src, dst, asad2, h2)
    out = _tc3(acc2.reshape(NCORE, NP, OUT), den2, h2, asad2, b2)
    return out[:N]


# speedup vs baseline: 50.6147x; 1.1577x over previous
import functools

import jax
import jax.numpy as jnp
from jax import lax
from jax.experimental import pallas as pl
from jax.experimental.pallas import tpu as pltpu
from jax.experimental.pallas import tpu_sc as plsc

N = 10000
NP = 10240
E = 320000
D = 128
H = 4
HID = 128
OUT = 128
NEG_SLOPE = 0.2

BN = 1024
CHUNK = 80
NSUB = 16
NCORE = 2
NPT = NP // NSUB
ZR = 40


def _tc1_body(x_ref, w1_ref, proj_ref, feats_ref, asad_ref):
    h = jnp.dot(x_ref[...], w1_ref[...], preferred_element_type=jnp.float32)
    for hh in range(H):
        feats_ref[hh] = h[:, hh * HID:(hh + 1) * HID]
    asad_ref[...] = lax.dot_general(
        proj_ref[...], h, (((1,), (1,)), ((), ())),
        preferred_element_type=jnp.float32)


def _tc1(x, W1, proj1):
    return pl.pallas_call(
        _tc1_body,
        grid=(NP // BN,),
        in_specs=[
            pl.BlockSpec((BN, D), lambda i: (i, 0)),
            pl.BlockSpec((D, H * HID), lambda i: (0, 0)),
            pl.BlockSpec((2 * H, H * HID), lambda i: (0, 0)),
        ],
        out_specs=[
            pl.BlockSpec((H, BN, HID), lambda i: (0, i, 0)),
            pl.BlockSpec((2 * H, BN), lambda i: (0, i)),
        ],
        out_shape=[
            jax.ShapeDtypeStruct((H, NP, HID), jnp.float32),
            jax.ShapeDtypeStruct((2 * H, NP), jnp.float32),
        ],
    )(x, W1, proj1)


def _tc2_body(acc_ref, den_ref, feats_ref, asad_ref, b1_ref, w2_ref,
              proj2_ref, h2_ref, asad2_ref):
    cols = []
    for hh in range(H):
        t = asad_ref[hh] + asad_ref[H + hh]
        p = jnp.exp(jnp.where(t >= 0, t, t * NEG_SLOPE))
        num = acc_ref[hh] + p[:, None] * feats_ref[hh]
        den = den_ref[hh] + p + 1e-16
        cols.append(num / den[:, None] + b1_ref[hh * HID:(hh + 1) * HID][None, :])
    act = jnp.maximum(jnp.concatenate(cols, axis=-1), 0.0)
    h2 = jnp.dot(act, w2_ref[...], preferred_element_type=jnp.float32)
    h2_ref[...] = h2
    asad2_ref[...] = lax.dot_general(
        proj2_ref[...], h2, (((1,), (1,)), ((), ())),
        preferred_element_type=jnp.float32)


def _tc2(acc1, den1, feats1, asad1, b1, W2, proj2):
    return pl.pallas_call(
        _tc2_body,
        grid=(NP // BN,),
        in_specs=[
            pl.BlockSpec((H, BN, HID), lambda i: (0, i, 0)),
            pl.BlockSpec((2 * H, BN), lambda i: (0, i)),
            pl.BlockSpec((H, BN, HID), lambda i: (0, i, 0)),
            pl.BlockSpec((2 * H, BN), lambda i: (0, i)),
            pl.BlockSpec((H * HID,), lambda i: (0,)),
            pl.BlockSpec((H * HID, OUT), lambda i: (0, 0)),
            pl.BlockSpec((2 * H, OUT), lambda i: (0, 0)),
        ],
        out_specs=[
            pl.BlockSpec((BN, OUT), lambda i: (i, 0)),
            pl.BlockSpec((2 * H, BN), lambda i: (0, i)),
        ],
        out_shape=[
            jax.ShapeDtypeStruct((NP, OUT), jnp.float32),
            jax.ShapeDtypeStruct((2 * H, NP), jnp.float32),
        ],
    )(acc1, den1, feats1, asad1, b1, W2, proj2)


def _tc3_body(acc_ref, den_ref, h2_ref, asad2_ref, b2_ref, out_ref):
    t = asad2_ref[0] + asad2_ref[1]
    p = jnp.exp(jnp.where(t >= 0, t, t * NEG_SLOPE))
    num = acc_ref[0] + acc_ref[1] + p[:, None] * h2_ref[...]
    den = den_ref[0] + den_ref[1] + p + 1e-16
    out_ref[...] = num / den[:, None] + b2_ref[...][None, :]


def _tc3(acc2, den2, h2, asad2, b2):
    return pl.pallas_call(
        _tc3_body,
        grid=(NP // BN,),
        in_specs=[
            pl.BlockSpec((NCORE, BN, OUT), lambda i: (0, i, 0)),
            pl.BlockSpec((2 * H, BN), lambda i: (0, i)),
            pl.BlockSpec((BN, OUT), lambda i: (i, 0)),
            pl.BlockSpec((2 * H, BN), lambda i: (0, i)),
            pl.BlockSpec((OUT,), lambda i: (0,)),
        ],
        out_specs=pl.BlockSpec((BN, OUT), lambda i: (i, 0)),
        out_shape=jax.ShapeDtypeStruct((NP, OUT), jnp.float32),
    )(acc2, den2, h2, asad2, b2)


_MESH = plsc.VectorSubcoreMesh(
    core_axis_name="c", subcore_axis_name="s", num_cores=NCORE,
    num_subcores=NSUB)

_BUF = [
    pltpu.VMEM((CHUNK,), jnp.int32),
    pltpu.VMEM((CHUNK,), jnp.int32),
    pltpu.VMEM((CHUNK,), jnp.int32),
    pltpu.VMEM((CHUNK,), jnp.float32),
    pltpu.VMEM((CHUNK, HID), jnp.float32),
    pltpu.VMEM((CHUNK,), jnp.int32),
    pltpu.SemaphoreType.DMA,
    pltpu.SemaphoreType.DMA,
    pltpu.SemaphoreType.DMA,
    pltpu.SemaphoreType.DMA,
]
_SC_SCRATCH = [
    pltpu.VMEM_SHARED((NP, HID), jnp.float32),
    pltpu.VMEM_SHARED((NP,), jnp.float32),
    pltpu.VMEM((NP,), jnp.float32),
    pltpu.VMEM((NP,), jnp.float32),
    pltpu.VMEM((ZR, HID), jnp.float32),
    pltpu.VMEM((NPT,), jnp.float32),
    _BUF,
    _BUF,
]


def _zero_bufs(zb, zden):
    def zrow(r, carry):
        for cc in range(HID // 16):
            zb[r, pl.ds(cc * 16, 16)] = jnp.zeros((16,), jnp.float32)
        return carry
    lax.fori_loop(0, ZR, zrow, 0)

    def zd(r, carry):
        zden[pl.ds(r * 16, 16)] = jnp.zeros((16,), jnp.float32)
        return carry
    lax.fori_loop(0, NPT // 16, zd, 0)


def _zero_acc(s, acc_s, den_s, zb, zden):
    base_n = s * NPT
    for z in range(NPT // ZR):
        pltpu.sync_copy(zb, acc_s.at[pl.ds(base_n + z * ZR, ZR)])
    pltpu.sync_copy(zden, den_s.at[pl.ds(s * NPT, NPT)])


def _edge_chunks(n_chunks, ebase, head_off, src_ref, dst_ref, feats_ref,
                 as_v, ad_v, bufs, acc_s, den_s):
    def idx_load(i, b, sync):
        srcb, dstb = bufs[b][0], bufs[b][1]
        isem = bufs[b][8]
        eb = ebase + i * CHUNK
        if sync:
            pltpu.sync_copy(src_ref.at[pl.ds(eb, CHUNK)], srcb)
            pltpu.sync_copy(dst_ref.at[pl.ds(eb, CHUNK)], dstb)
        else:
            pltpu.async_copy(src_ref.at[pl.ds(eb, CHUNK)], srcb, isem)
            pltpu.async_copy(dst_ref.at[pl.ds(eb, CHUNK)], dstb, isem)

    def stage_a(i, b, first):
        srcb, dstb, gidx, wv, rows, sdst, gsem, ssem, isem, dsem = bufs[b]
        if not first:
            pltpu.make_async_copy(src_ref.at[pl.ds(0, CHUNK)], srcb,
                                  isem).wait()
            pltpu.make_async_copy(dst_ref.at[pl.ds(0, CHUNK)], dstb,
                                  isem).wait()
            pltpu.make_async_copy(wv, den_s.at[sdst], dsem).wait()

        def wbody(kk, carry2):
            off = kk * 16
            sv = srcb[pl.ds(off, 16)]
            dv = dstb[pl.ds(off, 16)]
            t = plsc.load_gather(as_v, [sv]) + plsc.load_gather(ad_v, [dv])
            t = jnp.where(t >= 0, t, t * NEG_SLOPE)
            wv[pl.ds(off, 16)] = jnp.exp(t)
            gidx[pl.ds(off, 16)] = sv + head_off
            return carry2
        lax.fori_loop(0, CHUNK // 16, wbody, 0)
        if not first:
            pltpu.make_async_copy(rows, acc_s.at[sdst], ssem).wait()
        pltpu.async_copy(feats_ref.at[gidx], rows, gsem)

    def stage_b(i, b):
        srcb, dstb, gidx, wv, rows, sdst, gsem, ssem, isem, dsem = bufs[b]
        pltpu.make_async_copy(feats_ref.at[gidx], rows, gsem).wait()
        for kk in range(CHUNK // 16):
            sdst[pl.ds(kk * 16, 16)] = dstb[pl.ds(kk * 16, 16)]

        @pl.when(i + 2 < n_chunks)
        def _():
            idx_load(i + 2, b, False)

        def mbody(j):
            wj = plsc.load_gather(wv, [jnp.full((16,), j, jnp.int32)])
            for cc in range(HID // 16):
                rows[j, pl.ds(cc * 16, 16)] = rows[j, pl.ds(cc * 16, 16)] * wj
        plsc.parallel_loop(0, CHUNK, 1, unroll=8)(mbody)

        pltpu.async_copy(rows, acc_s.at[sdst], ssem, add=True)
        pltpu.async_copy(wv, den_s.at[sdst], dsem, add=True)

    idx_load(0, 0, True)
    idx_load(1, 1, True)
    stage_a(0, 0, True)
    stage_a(1, 1, True)

    def loop_body(i2, carry):
        i = i2 * 2
        stage_b(i, 0)

        @pl.when(i + 2 < n_chunks)
        def _():
            stage_a(i + 2, 0, False)

        @pl.when(i + 1 < n_chunks)
        def _():
            stage_b(i + 1, 1)

            @pl.when(i + 3 < n_chunks)
            def _():
                stage_a(i + 3, 1, False)
        return carry
    lax.fori_loop(0, (n_chunks + 1) // 2, loop_body, 0)
    for b in range(2):
        srcb, dstb, gidx, wv, rows, sdst, gsem, ssem, isem, dsem = bufs[b]
        pltpu.make_async_copy(rows, acc_s.at[sdst], ssem).wait()
        pltpu.make_async_copy(wv, den_s.at[sdst], dsem).wait()


@functools.partial(
    pl.kernel,
    out_type=(jax.ShapeDtypeStruct((H * NP, HID), jnp.float32),
              jax.ShapeDtypeStruct((2 * H, NP), jnp.float32)),
    mesh=_MESH,
    scratch_types=_SC_SCRATCH,
    compiler_params=pltpu.CompilerParams(needs_layout_passes=False),
)
def _sc1(src_ref, dst_ref, asad_ref, feats_ref, accs_ref, dens_ref,
         acc_s, den_s, as_v, ad_v, zb, zden, buf0, buf1):
    c = lax.axis_index("c")
    s = lax.axis_index("s")
    _zero_bufs(zb, zden)
    eps = E // NSUB
    for hh in range(2):
        head = c * 2 + hh
        _zero_acc(s, acc_s, den_s, zb, zden)
        pltpu.sync_copy(asad_ref.at[head], as_v)
        pltpu.sync_copy(asad_ref.at[H + head], ad_v)
        plsc.subcore_barrier()
        _edge_chunks(eps // CHUNK, s * eps, head * NP, src_ref, dst_ref,
                     feats_ref, as_v, ad_v, (buf0, buf1), acc_s, den_s)
        plsc.subcore_barrier()
        base_n = s * NPT
        pltpu.sync_copy(acc_s.at[pl.ds(base_n, NPT)],
                        accs_ref.at[pl.ds(head * NP + base_n, NPT)])
        @pl.when(s == 0)
        def _():
            pltpu.sync_copy(den_s.at[pl.ds(0, NP)], dens_ref.at[head])
        plsc.subcore_barrier()


@functools.partial(
    pl.kernel,
    out_type=(jax.ShapeDtypeStruct((NCORE * NP, OUT), jnp.float32),
              jax.ShapeDtypeStruct((2 * H, NP), jnp.float32)),
    mesh=_MESH,
    scratch_types=_SC_SCRATCH,
    compiler_params=pltpu.CompilerParams(needs_layout_passes=False),
)
def _sc2(src_ref, dst_ref, asad_ref, feats_ref, accs_ref, dens_ref,
         acc_s, den_s, as_v, ad_v, zb, zden, buf0, buf1):
    c = lax.axis_index("c")
    s = lax.axis_index("s")
    _zero_bufs(zb, zden)
    eps = E // (NCORE * NSUB)
    _zero_acc(s, acc_s, den_s, zb, zden)
    pltpu.sync_copy(asad_ref.at[0], as_v)
    pltpu.sync_copy(asad_ref.at[1], ad_v)
    plsc.subcore_barrier()
    _edge_chunks(eps // CHUNK, (c * NSUB + s) * eps, 0, src_ref, dst_ref,
                 feats_ref, as_v, ad_v, (buf0, buf1), acc_s, den_s)
    plsc.subcore_barrier()
    base_n = s * NPT
    pltpu.sync_copy(acc_s.at[pl.ds(base_n, NPT)],
                    accs_ref.at[pl.ds(c * NP + base_n, NPT)])
    @pl.when(s == 0)
    def _():
        pltpu.sync_copy(den_s.at[pl.ds(0, NP)], dens_ref.at[c])


def kernel(x, edge_index, W1, a_src1, a_dst1, b1, W2, a_src2, a_dst2, b2):
    src = edge_index[0]
    dst = edge_index[1]
    x_p = jnp.pad(x, ((0, NP - N), (0, 0)))
    eye = jnp.eye(H, dtype=jnp.float32)
    proj1 = jnp.concatenate(
        [(eye[:, :, None] * a_src1[None, :, :]).reshape(H, H * HID),
         (eye[:, :, None] * a_dst1[None, :, :]).reshape(H, H * HID)], axis=0)
    proj2 = jnp.concatenate(
        [a_src2, a_dst2, jnp.zeros((2 * H - 2, OUT), jnp.float32)], axis=0)
    feats1, asad1 = _tc1(x_p, W1, proj1)
    acc1, den1 = _sc1(src, dst, asad1, feats1.reshape(H * NP, HID))
    h2, asad2 = _tc2(acc1.reshape(H, NP, HID), den1, feats1, asad1, b1, W2,
                     proj2)
    acc2, den2 = _sc2(src, dst, asad2, h2)
    out = _tc3(acc2.reshape(NCORE, NP, OUT), den2, h2, asad2, b2)
    return out[:N]
